# Initial kernel scaffold; baseline (speedup 1.0000x reference)
import jax
import jax.numpy as jnp
from jax.experimental import pallas as pl


def kernel(features, rois):
    raise NotImplementedError("write your pallas kernel here")


# speedup vs baseline: 2972.9979x; 2972.9979x over previous
import jax
import jax.numpy as jnp
from jax.experimental import pallas as pl
from jax.experimental.pallas import tpu as pltpu

_PH, _PW = 7, 7
_SCALE = 0.0625
_MAXC = 32
_NEG = float(jnp.finfo(jnp.float32).min)


def _roi_kernel(rois_ref, f_ref, o_ref):
    b = pl.program_id(0)
    r = pl.program_id(1)
    H = f_ref.shape[1]
    W = f_ref.shape[2]

    cx = rois_ref[b, r, 1].astype(jnp.float32)
    cy = rois_ref[b, r, 2].astype(jnp.float32)
    cw = rois_ref[b, r, 3].astype(jnp.float32)
    ch = rois_ref[b, r, 4].astype(jnp.float32)
    x = _SCALE * cx
    y = _SCALE * cy
    w = jnp.maximum(_SCALE * cw, 1.0)
    h = jnp.maximum(_SCALE * ch, 1.0)
    y = jnp.where(y >= H, H - h, y)
    x = jnp.where(x >= W, W - w, x)
    y0 = jnp.floor(y).astype(jnp.int32)
    x0 = jnp.floor(x).astype(jnp.int32)
    hc = jnp.maximum(jnp.minimum(jnp.floor(y + h).astype(jnp.int32), H) - y0, 1)
    wc = jnp.maximum(jnp.minimum(jnp.floor(x + w).astype(jnp.int32), W) - x0, 1)

    y0c = jnp.clip(y0, 0, H - _MAXC)
    x0c = jnp.clip(x0, 0, W - _MAXC)
    dy = y0 - y0c
    dx = x0 - x0c

    crop = f_ref[0, pl.ds(y0c, _MAXC), pl.ds(x0c, _MAXC), :]

    dw = jax.lax.broadcasted_iota(jnp.int32, (1, _MAXC, 1), 1) - dx
    dh = jax.lax.broadcasted_iota(jnp.int32, (_MAXC, 1, 1), 0) - dy

    tmps = []
    for j in range(_PW):
        ws = (j * wc) // _PW
        we = -((-(j + 1) * wc) // _PW)
        m = (dw >= ws) & (dw < we)
        tmps.append(jnp.where(m, crop, _NEG).max(axis=1))
    tmp = jnp.stack(tmps, axis=1)

    rows = []
    for i in range(_PH):
        bs = (i * hc) // _PH
        be = -((-(i + 1) * hc) // _PH)
        m = (dh >= bs) & (dh < be)
        rows.append(jnp.where(m, tmp, _NEG).max(axis=0))
    out = jnp.stack(rows, axis=0)
    o_ref[0, 0] = out.reshape(_PH * _PW, out.shape[-1])


def kernel(features, rois):
    B, C, H, W = features.shape
    R = rois.shape[1]
    f_t = jnp.transpose(features, (0, 2, 3, 1))
    rois32 = rois.astype(jnp.int32)
    out = pl.pallas_call(
        _roi_kernel,
        out_shape=jax.ShapeDtypeStruct((B, R, _PH * _PW, C), jnp.float32),
        grid_spec=pltpu.PrefetchScalarGridSpec(
            num_scalar_prefetch=1,
            grid=(B, R),
            in_specs=[pl.BlockSpec((1, H, W, C), lambda b, r, rois_s: (b, 0, 0, 0))],
            out_specs=pl.BlockSpec((1, 1, _PH * _PW, C),
                                   lambda b, r, rois_s: (b, r, 0, 0)),
        ),
        compiler_params=pltpu.CompilerParams(
            dimension_semantics=("parallel", "arbitrary"),
        ),
        name="roi_maxpool",
    )(rois32, f_t)
    return out.transpose(0, 1, 3, 2).reshape(B, R, C, _PH, _PW)


# speedup vs baseline: 5150.6730x; 1.7325x over previous
import jax
import jax.numpy as jnp
from jax.experimental import pallas as pl
from jax.experimental.pallas import tpu as pltpu

_PH, _PW = 7, 7
_SCALE = 0.0625
_BINW = 6
_ROWS = 40
_NEG = float(jnp.finfo(jnp.float32).min)


def _roi_kernel(rois_ref, f_ref, o_ref):
    b = pl.program_id(0)
    r = pl.program_id(1)
    W = f_ref.shape[1]
    H = f_ref.shape[2]

    cx = rois_ref[b, r, 1].astype(jnp.float32)
    cy = rois_ref[b, r, 2].astype(jnp.float32)
    cw = rois_ref[b, r, 3].astype(jnp.float32)
    ch = rois_ref[b, r, 4].astype(jnp.float32)
    x = _SCALE * cx
    y = _SCALE * cy
    w = jnp.maximum(_SCALE * cw, 1.0)
    h = jnp.maximum(_SCALE * ch, 1.0)
    y = jnp.where(y >= H, H - h, y)
    x = jnp.where(x >= W, W - w, x)
    y0 = jnp.floor(y).astype(jnp.int32)
    x0 = jnp.floor(x).astype(jnp.int32)
    hc = jnp.maximum(jnp.minimum(jnp.floor(y + h).astype(jnp.int32), H) - y0, 1)
    wc = jnp.maximum(jnp.minimum(jnp.floor(x + w).astype(jnp.int32), W) - x0, 1)

    y0a = jnp.minimum((y0 // 8) * 8, H - _ROWS)
    y0a = pl.multiple_of(y0a, 8)
    dy = y0 - y0a

    tmps = []
    for j in range(_PW):
        s = (j * wc) // _PW
        e = -((-(j + 1) * wc) // _PW)
        last = e - 1
        acc = None
        for t in range(_BINW):
            wt = jnp.maximum(x0 + jnp.minimum(s + t, last), 0)
            col = f_ref[0, wt, pl.ds(y0a, _ROWS), :]
            acc = col if acc is None else jnp.maximum(acc, col)
        acc = jnp.where(e > s, acc, _NEG)
        tmps.append(acc)

    dh = jax.lax.broadcasted_iota(jnp.int32, (_ROWS, 1), 0) - dy
    for i in range(_PH):
        bs = (i * hc) // _PH
        be = -((-(i + 1) * hc) // _PH)
        m = (dh >= bs) & (dh < be)
        row = jnp.stack(
            [jnp.where(m, tmps[j], _NEG).max(axis=0) for j in range(_PW)],
            axis=0)
        o_ref[0, 0, _PW * i:_PW * (i + 1), :] = row


def kernel(features, rois):
    B, C, H, W = features.shape
    R = rois.shape[1]
    f_t = jnp.transpose(features, (0, 3, 2, 1))
    rois32 = rois.astype(jnp.int32)
    out = pl.pallas_call(
        _roi_kernel,
        out_shape=jax.ShapeDtypeStruct((B, R, _PH * _PW, C), jnp.float32),
        grid_spec=pltpu.PrefetchScalarGridSpec(
            num_scalar_prefetch=1,
            grid=(B, R),
            in_specs=[pl.BlockSpec((1, W, H, C), lambda b, r, rois_s: (b, 0, 0, 0))],
            out_specs=pl.BlockSpec((1, 1, _PH * _PW, C),
                                   lambda b, r, rois_s: (b, r, 0, 0)),
        ),
        compiler_params=pltpu.CompilerParams(
            dimension_semantics=("parallel", "arbitrary"),
        ),
        name="roi_maxpool",
    )(rois32, f_t)
    return out.transpose(0, 1, 3, 2).reshape(B, R, C, _PH, _PW)


# speedup vs baseline: 10276.9250x; 1.9953x over previous
import jax
import jax.numpy as jnp
from jax.experimental import pallas as pl
from jax.experimental.pallas import tpu as pltpu

_PH, _PW = 7, 7
_SCALE = 0.0625
_BINW = 6
_ROWS = 40
_RB = 4
_NEG = float(jnp.finfo(jnp.float32).min)


def _roi_kernel(meta_ref, f_ref, o_ref):
    b = pl.program_id(0)
    rblk = pl.program_id(1)
    iot = jax.lax.broadcasted_iota(jnp.int32, (_ROWS, 1), 0)

    for u in range(_RB):
        rr = rblk * _RB + u
        y0a = pl.multiple_of(meta_ref[b, rr, 0], 8)
        rows_sl = pl.ds(y0a, _ROWS)

        tmps = []
        for j in range(_PW):
            base = 1 + _BINW * j
            acc = f_ref[0, meta_ref[b, rr, base], rows_sl, :]
            for t in range(1, _BINW):
                acc = jnp.maximum(
                    acc, f_ref[0, meta_ref[b, rr, base + t], rows_sl, :])
            acc = jnp.where(meta_ref[b, rr, 43 + j] > 0, acc, _NEG)
            tmps.append(acc)

        for i in range(_PH):
            m = (iot >= meta_ref[b, rr, 50 + i]) & (iot < meta_ref[b, rr, 57 + i])
            row = jnp.stack(
                [jnp.where(m, tmps[j], _NEG).max(axis=0) for j in range(_PW)],
                axis=0)
            o_ref[0, u, _PW * i:_PW * (i + 1), :] = row


def _make_meta(rois, H, W):
    r = rois.astype(jnp.float32)
    x = _SCALE * r[..., 1]
    y = _SCALE * r[..., 2]
    w = jnp.maximum(_SCALE * r[..., 3], 1.0)
    h = jnp.maximum(_SCALE * r[..., 4], 1.0)
    y = jnp.where(y >= H, H - h, y)
    x = jnp.where(x >= W, W - w, x)
    y0 = jnp.floor(y).astype(jnp.int32)
    x0 = jnp.floor(x).astype(jnp.int32)
    hc = jnp.maximum(jnp.minimum(jnp.floor(y + h).astype(jnp.int32), H) - y0, 1)
    wc = jnp.maximum(jnp.minimum(jnp.floor(x + w).astype(jnp.int32), W) - x0, 1)

    y0a = jnp.clip((y0 // 8) * 8, 0, H - _ROWS)
    dy = y0 - y0a

    jj = jnp.arange(_PW)
    ws = (jj * wc[..., None]) // _PW
    we = -((-(jj + 1) * wc[..., None]) // _PW)
    tt = jnp.arange(_BINW)
    wabs = x0[..., None, None] + jnp.minimum(ws[..., None] + tt, we[..., None] - 1)
    wabs = jnp.clip(wabs, 0, W - 1)
    wflag = (we > ws).astype(jnp.int32)

    ii = jnp.arange(_PH)
    bs = (ii * hc[..., None]) // _PH
    be = -((-(ii + 1) * hc[..., None]) // _PH)
    lo = bs + dy[..., None]
    hi = be + dy[..., None]

    B, R = rois.shape[:2]
    pad = jnp.zeros((B, R, 7), jnp.int32)
    return jnp.concatenate(
        [y0a[..., None],
         wabs.reshape(B, R, _PW * _BINW),
         wflag, lo, hi, pad], axis=-1)


def kernel(features, rois):
    B, C, H, W = features.shape
    R = rois.shape[1]
    f_t = jnp.transpose(features, (0, 3, 2, 1))
    meta = _make_meta(rois, H, W)
    out = pl.pallas_call(
        _roi_kernel,
        out_shape=jax.ShapeDtypeStruct((B, R, _PH * _PW, C), jnp.float32),
        grid_spec=pltpu.PrefetchScalarGridSpec(
            num_scalar_prefetch=1,
            grid=(B, R // _RB),
            in_specs=[pl.BlockSpec((1, W, H, C), lambda b, r, meta_s: (b, 0, 0, 0))],
            out_specs=pl.BlockSpec((1, _RB, _PH * _PW, C),
                                   lambda b, r, meta_s: (b, r, 0, 0)),
        ),
        compiler_params=pltpu.CompilerParams(
            dimension_semantics=("parallel", "arbitrary"),
        ),
        name="roi_maxpool",
    )(meta, f_t)
    return out.transpose(0, 1, 3, 2).reshape(B, R, C, _PH, _PW)


# speedup vs baseline: 11700.3845x; 1.1385x over previous
import jax
import jax.numpy as jnp
from jax.experimental import pallas as pl
from jax.experimental.pallas import tpu as pltpu

_PH, _PW = 7, 7
_SCALE = 0.0625
_ROWS = 40
_RB = 8
_NEG = float(jnp.finfo(jnp.float32).min)


def _roi_kernel(meta_ref, f_ref, o_ref, ptab_ref):
    b = pl.program_id(0)
    rblk = pl.program_id(1)
    W = f_ref.shape[1]

    @pl.when(rblk == 0)
    def _build_tables():
        ptab_ref[0:W] = f_ref[0]
        ptab_ref[W:2 * W - 1] = jnp.maximum(f_ref[0, 0:W - 1], f_ref[0, 1:W])
        ptab_ref[2 * W - 1] = f_ref[0, W - 1]
        ptab_ref[2 * W:3 * W - 2] = jnp.maximum(ptab_ref[W:2 * W - 2],
                                                ptab_ref[W + 2:2 * W])
        ptab_ref[3 * W - 2:3 * W] = ptab_ref[2 * W - 2:2 * W]
        ptab_ref[3 * W:4 * W - 4] = jnp.maximum(ptab_ref[2 * W:3 * W - 4],
                                                ptab_ref[2 * W + 4:3 * W])
        ptab_ref[4 * W - 4:4 * W] = ptab_ref[3 * W - 4:3 * W]

    iot = jax.lax.broadcasted_iota(jnp.int32, (_ROWS, 1), 0)

    for u in range(_RB):
        rr = rblk * _RB + u
        y0a = pl.multiple_of(meta_ref[b, rr, 0], 8)
        rows_sl = pl.ds(y0a, _ROWS)

        tmps = []
        for j in range(_PW):
            acc = jnp.maximum(ptab_ref[meta_ref[b, rr, 1 + j], rows_sl, :],
                              ptab_ref[meta_ref[b, rr, 8 + j], rows_sl, :])
            acc = jnp.where(meta_ref[b, rr, 15 + j] > 0, acc, _NEG)
            tmps.append(acc)

        for i in range(_PH):
            m = (iot >= meta_ref[b, rr, 22 + i]) & (iot < meta_ref[b, rr, 29 + i])
            row = jnp.stack(
                [jnp.where(m, tmps[j], _NEG).max(axis=0) for j in range(_PW)],
                axis=0)
            o_ref[0, u, _PW * i:_PW * (i + 1), :] = row


def _make_meta(rois, H, W):
    r = rois.astype(jnp.float32)
    x = _SCALE * r[..., 1]
    y = _SCALE * r[..., 2]
    w = jnp.maximum(_SCALE * r[..., 3], 1.0)
    h = jnp.maximum(_SCALE * r[..., 4], 1.0)
    y = jnp.where(y >= H, H - h, y)
    x = jnp.where(x >= W, W - w, x)
    y0 = jnp.floor(y).astype(jnp.int32)
    x0 = jnp.floor(x).astype(jnp.int32)
    hc = jnp.maximum(jnp.minimum(jnp.floor(y + h).astype(jnp.int32), H) - y0, 1)
    wc = jnp.maximum(jnp.minimum(jnp.floor(x + w).astype(jnp.int32), W) - x0, 1)

    y0a = jnp.clip((y0 // 8) * 8, 0, H - _ROWS)
    dy = y0 - y0a

    jj = jnp.arange(_PW)
    ws = (jj * wc[..., None]) // _PW
    we = -((-(jj + 1) * wc[..., None]) // _PW)
    L = we - ws
    k = ((L >= 2).astype(jnp.int32) + (L >= 4).astype(jnp.int32)
         + (L >= 8).astype(jnp.int32))
    pw2 = jnp.left_shift(1, k)
    ia = k * W + jnp.clip(x0[..., None] + ws, 0, W - 1)
    ib = k * W + jnp.clip(x0[..., None] + we - pw2, 0, W - 1)
    wflag = (we > ws).astype(jnp.int32)

    ii = jnp.arange(_PH)
    bs = (ii * hc[..., None]) // _PH
    be = -((-(ii + 1) * hc[..., None]) // _PH)
    lo = bs + dy[..., None]
    hi = be + dy[..., None]

    B, R = rois.shape[:2]
    pad = jnp.zeros((B, R, 4), jnp.int32)
    return jnp.concatenate(
        [y0a[..., None], ia, ib, wflag, lo, hi, pad], axis=-1)


def kernel(features, rois):
    B, C, H, W = features.shape
    R = rois.shape[1]
    f_t = jnp.transpose(features, (0, 3, 2, 1))
    meta = _make_meta(rois, H, W)
    out = pl.pallas_call(
        _roi_kernel,
        out_shape=jax.ShapeDtypeStruct((B, R, _PH * _PW, C), jnp.float32),
        grid_spec=pltpu.PrefetchScalarGridSpec(
            num_scalar_prefetch=1,
            grid=(B, R // _RB),
            in_specs=[pl.BlockSpec((1, W, H, C), lambda b, r, meta_s: (b, 0, 0, 0))],
            out_specs=pl.BlockSpec((1, _RB, _PH * _PW, C),
                                   lambda b, r, meta_s: (b, r, 0, 0)),
            scratch_shapes=[pltpu.VMEM((4 * W, H, C), jnp.float32)],
        ),
        compiler_params=pltpu.CompilerParams(
            dimension_semantics=("parallel", "arbitrary"),
        ),
        name="roi_maxpool",
    )(meta, f_t)
    return out.transpose(0, 1, 3, 2).reshape(B, R, C, _PH, _PW)
